# baseline (speedup 1.0000x reference)
import functools

import jax
import jax.numpy as jnp
from jax import lax
from jax.experimental import pallas as pl
from jax.experimental.pallas import tpu as pltpu
from jax.experimental.pallas import tpu_sc as plsc

B, S, D = 4, 8192, 2048
L = 16
W = 16
WPB = W // B
CHUNK = S // WPB
UNROLL = 8


@functools.partial(
    pl.kernel,
    mesh=plsc.VectorSubcoreMesh(core_axis_name="c", subcore_axis_name="s",
                                num_cores=1),
    compiler_params=pltpu.CompilerParams(needs_layout_passes=False),
    out_type=jax.ShapeDtypeStruct((B, D), jnp.float32),
    scratch_types=[
        pltpu.VMEM((CHUNK,), jnp.int32),
        pltpu.VMEM((D,), jnp.float32),
        pltpu.SMEM((1,), jnp.int32),
    ],
)
def _gather_last(logits_hbm, tok_hbm, out_hbm, tok_v, row_v, cnt_s):
    w = lax.axis_index("s")
    b = w // WPB
    c = w % WPB

    @pl.when(w < B)
    def _():
        cnt_s[0] = 0

    pltpu.sync_copy(tok_hbm.at[b, pl.ds(c * CHUNK, CHUNK)], tok_v)

    def body(i, acc):
        for j in range(UNROLL):
            x = tok_v[pl.ds((i * UNROLL + j) * L, L)]
            acc = acc + jnp.where(x != 0, 1, 0).astype(jnp.int32)
        return acc

    acc = lax.fori_loop(0, CHUNK // (L * UNROLL), body,
                        jnp.zeros((L,), jnp.int32))
    partial = jnp.sum(acc)
    plsc.subcore_barrier()
    plsc.fetch_and_add(cnt_s.at[0], partial, subcore_id=b)
    plsc.subcore_barrier()

    @pl.when(w < B)
    def _():
        row = jnp.maximum(cnt_s[0] - 1, 0)
        pltpu.sync_copy(logits_hbm.at[w, row], row_v)
        pltpu.sync_copy(row_v, out_hbm.at[w])


def kernel(logits, token_seq):
    return _gather_last(logits, token_seq.astype(jnp.int32))
